# Initial kernel scaffold; baseline (speedup 1.0000x reference)
import jax
import jax.numpy as jnp
from jax.experimental import pallas as pl


def kernel(x, edge_index_a, edge_index_b, W1_a, b1_a, W1_b, b1_b, W2_a, b2_a, W2_b, b2_b):
    raise NotImplementedError("write your pallas kernel here")


# speedup vs baseline: 3.1093x; 3.1093x over previous
import functools

import jax
import jax.numpy as jnp
import numpy as np
from jax import lax
from jax.experimental import pallas as pl
from jax.experimental.pallas import tpu as pltpu
from jax.experimental.pallas import tpu_sc as plsc

NC = 2
NS = 16
CHUNK = 128
GRP = 16


def _sc_agg(feat, src_all, dst_all, zrows, ones_c, want_counts):
    NPAD, D = feat.shape
    CH = src_all.shape[0] // (NC * NS)
    rpt = NPAD // NS

    full, tail = rpt // CHUNK, rpt % CHUNK
    pieces = [(k * CHUNK, CHUNK) for k in range(full)]
    if tail:
        pieces.append((full * CHUNK, tail))
    NW = len(pieces)
    NWP = -(-NW // 8) * 8

    own_np = np.zeros((NS * NWP, CHUNK), np.int32)
    lane = np.arange(CHUNK)
    for s in range(NS):
        for k, (off, ln) in enumerate(pieces):
            own_np[s * NWP + k] = s * rpt + off + np.minimum(lane, ln - 1)
    own_idx = jnp.asarray(own_np)

    mesh = plsc.VectorSubcoreMesh(
        core_axis_name="c", subcore_axis_name="s", num_cores=NC, num_subcores=NS)

    out_type = [jax.ShapeDtypeStruct((NC * NPAD, D), jnp.float32)]
    if want_counts:
        out_type.append(jax.ShapeDtypeStruct((NC * NPAD, D), jnp.float32))
    scratch = [
        pltpu.VMEM_SHARED((NPAD, D), jnp.float32),
        pltpu.VMEM((GRP, CHUNK), jnp.int32),
        pltpu.VMEM((GRP, CHUNK), jnp.int32),
        pltpu.VMEM((NWP, CHUNK), jnp.int32),
        pltpu.VMEM((CHUNK, D), jnp.float32),
        pltpu.SemaphoreType.DMA,
    ]

    @functools.partial(pl.kernel, out_type=tuple(out_type), mesh=mesh,
                       scratch_types=scratch)
    def run(*refs):
        if want_counts:
            (feat_h, src_h, dst_h, own_h, zr_h, on_h,
             sums_h, cnts_h, acc, src_v, dst_v, own_v, rows_v, sem) = refs
        else:
            (feat_h, src_h, dst_h, own_h, zr_h, on_h,
             sums_h, acc, src_v, dst_v, own_v, rows_v, sem) = refs
        c = lax.axis_index("c")
        s = lax.axis_index("s")
        r0 = s * rpt
        idx0 = (c * NS + s) * CH
        o0 = c * NPAD + r0

        pltpu.sync_copy(own_h.at[pl.ds(s * NWP, NWP)], own_v)

        def zero_acc():
            pltpu.sync_copy(zr_h, rows_v)
            for k in range(NW):
                pltpu.sync_copy(rows_v, acc.at[own_v.at[k]])

        def readback(dst_hbm):
            for k, (off, ln) in enumerate(pieces):
                pltpu.async_copy(acc.at[own_v.at[k]], rows_v, sem).wait()
                pltpu.sync_copy(rows_v.at[pl.ds(0, ln)],
                                dst_hbm.at[pl.ds(o0 + off, ln)])

        if want_counts:
            zero_acc()
            pltpu.sync_copy(on_h, rows_v)
            plsc.subcore_barrier()

            @pl.loop(0, CH // GRP)
            def _cgroup(g):
                pltpu.sync_copy(dst_h.at[pl.ds(idx0 + g * GRP, GRP)], dst_v)

                @pl.loop(0, GRP)
                def _cchunk(j):
                    pltpu.sync_copy(rows_v, acc.at[dst_v.at[j]], add=True)

            plsc.subcore_barrier()
            readback(cnts_h)

        zero_acc()
        plsc.subcore_barrier()

        @pl.loop(0, CH // GRP)
        def _group(g):
            pltpu.sync_copy(src_h.at[pl.ds(idx0 + g * GRP, GRP)], src_v)
            pltpu.sync_copy(dst_h.at[pl.ds(idx0 + g * GRP, GRP)], dst_v)

            @pl.loop(0, GRP)
            def _chunk(j):
                pltpu.async_copy(feat_h.at[src_v.at[j]], rows_v, sem).wait()
                pltpu.sync_copy(rows_v, acc.at[dst_v.at[j]], add=True)

        plsc.subcore_barrier()
        readback(sums_h)

    return run(feat, src_all, dst_all, own_idx, zrows, ones_c)


def _tc_layer(sums, cnts, Wa, ba, Wb, bb, relu):
    NPAD, D = sums.shape[1], sums.shape[2]

    def body(s_ref, c_ref, wa_ref, ba_ref, wb_ref, bb_ref, o_ref):
        ca = c_ref[0][:, 0:1]
        cb = c_ref[1][:, 0:1]
        ma = s_ref[0] / jnp.maximum(ca, 1.0)
        mb = s_ref[1] / jnp.maximum(cb, 1.0)
        h = (jnp.dot(ma, wa_ref[...], preferred_element_type=jnp.float32)
             + jnp.dot(mb, wb_ref[...], preferred_element_type=jnp.float32)
             + jnp.where(ca > 0, 1.0, 0.0) * ba_ref[...]
             + jnp.where(cb > 0, 1.0, 0.0) * bb_ref[...])
        if relu:
            h = jnp.where(h >= 0, h, 0.01 * h)
        o_ref[...] = h

    return pl.pallas_call(
        body,
        out_shape=jax.ShapeDtypeStruct((NPAD, D), jnp.float32),
    )(sums, cnts, Wa, ba.reshape(1, D), Wb, bb.reshape(1, D))


def kernel(x, edge_index_a, edge_index_b, W1_a, b1_a, W1_b, b1_b,
           W2_a, b2_a, W2_b, b2_b):
    N, D = x.shape
    E = edge_index_a.shape[1]
    NPAD = -(-(N + 1) // (NS * 8)) * (NS * 8)
    per_tile = -(-E // (NS * CHUNK * GRP)) * CHUNK * GRP
    CH = per_tile // CHUNK
    EPAD = per_tile * NS

    def pad_idx(v):
        return jnp.concatenate(
            [v, jnp.full((EPAD - E,), N, jnp.int32)]).reshape(NS * CH, CHUNK)

    src_all = jnp.concatenate([pad_idx(edge_index_a[0]), pad_idx(edge_index_b[0])])
    dst_all = jnp.concatenate([pad_idx(edge_index_a[1]), pad_idx(edge_index_b[1])])

    x_pad = jnp.pad(x, ((0, NPAD - N), (0, 0)))
    zrows = jnp.zeros((CHUNK, D), jnp.float32)
    ones_c = jnp.ones((CHUNK, D), jnp.float32)

    sums1, cnts = _sc_agg(x_pad, src_all, dst_all, zrows, ones_c, True)
    sums1 = sums1.reshape(NC, NPAD, D)
    cnts = cnts.reshape(NC, NPAD, D)
    h = _tc_layer(sums1, cnts, W1_a, b1_a, W1_b, b1_b, relu=True)
    res2 = _sc_agg(h, src_all, dst_all, zrows, ones_c, False)
    sums2 = (res2[0] if isinstance(res2, (tuple, list)) else res2)
    sums2 = sums2.reshape(NC, NPAD, D)
    out = _tc_layer(sums2, cnts, W2_a, b2_a, W2_b, b2_b, relu=False)
    return out[:N]


# speedup vs baseline: 3.3851x; 1.0887x over previous
import functools

import jax
import jax.numpy as jnp
import numpy as np
from jax import lax
from jax.experimental import pallas as pl
from jax.experimental.pallas import tpu as pltpu
from jax.experimental.pallas import tpu_sc as plsc

NC = 2
NS = 16
CHUNK = 128
GRP = 16


def _sc_agg(feat, src_all, dst_all, zrows, ones_c, want_counts):
    NPAD, D = feat.shape
    CH = src_all.shape[0] // (NC * NS)
    rpt = NPAD // NS

    full, tail = rpt // CHUNK, rpt % CHUNK
    pieces = [(k * CHUNK, CHUNK) for k in range(full)]
    if tail:
        pieces.append((full * CHUNK, tail))
    NW = len(pieces)
    NWP = -(-NW // 8) * 8

    own_np = np.zeros((NS * NWP, CHUNK), np.int32)
    lane = np.arange(CHUNK)
    for s in range(NS):
        for k, (off, ln) in enumerate(pieces):
            own_np[s * NWP + k] = s * rpt + off + np.minimum(lane, ln - 1)
    own_idx = jnp.asarray(own_np)

    mesh = plsc.VectorSubcoreMesh(
        core_axis_name="c", subcore_axis_name="s", num_cores=NC, num_subcores=NS)

    out_type = [jax.ShapeDtypeStruct((NC * NPAD, D), jnp.float32)]
    if want_counts:
        out_type.append(jax.ShapeDtypeStruct((NC * NPAD, D), jnp.float32))
    scratch = [
        pltpu.VMEM_SHARED((NPAD, D), jnp.float32),
        pltpu.VMEM((GRP, CHUNK), jnp.int32),
        pltpu.VMEM((GRP, CHUNK), jnp.int32),
        pltpu.VMEM((NWP, CHUNK), jnp.int32),
        pltpu.VMEM((2, CHUNK, D), jnp.float32),
        pltpu.SemaphoreType.DMA,
        pltpu.SemaphoreType.DMA,
    ]
    if want_counts:
        scratch.append(pltpu.SemaphoreType.DMA)
    FD = 8

    @functools.partial(pl.kernel, out_type=tuple(out_type), mesh=mesh,
                       scratch_types=scratch)
    def run(*refs):
        if want_counts:
            (feat_h, src_h, dst_h, own_h, zr_h, on_h,
             sums_h, cnts_h, acc, src_v, dst_v, own_v, rows_v, sem0, sem1,
             semc) = refs
        else:
            (feat_h, src_h, dst_h, own_h, zr_h, on_h,
             sums_h, acc, src_v, dst_v, own_v, rows_v, sem0, sem1) = refs
        c = lax.axis_index("c")
        s = lax.axis_index("s")
        r0 = s * rpt
        idx0 = (c * NS + s) * CH
        o0 = c * NPAD + r0
        sems = (sem0, sem1)

        pltpu.sync_copy(own_h.at[pl.ds(s * NWP, NWP)], own_v)

        def zero_acc():
            pltpu.sync_copy(zr_h, rows_v.at[0])
            for k in range(NW):
                pltpu.sync_copy(rows_v.at[0], acc.at[own_v.at[k]])

        def readback(dst_hbm):
            for k, (off, ln) in enumerate(pieces):
                pltpu.async_copy(acc.at[own_v.at[k]], rows_v.at[0], sem0).wait()
                pltpu.sync_copy(rows_v.at[0, pl.ds(0, ln)],
                                dst_hbm.at[pl.ds(o0 + off, ln)])

        if want_counts:
            zero_acc()
            pltpu.sync_copy(on_h, rows_v.at[0])
            plsc.subcore_barrier()

            @pl.loop(0, CH // GRP)
            def _cgroup(g):
                pltpu.sync_copy(dst_h.at[pl.ds(idx0 + g * GRP, GRP)], dst_v)

                @pl.loop(0, GRP // FD)
                def _half(h):
                    for t in range(FD):
                        pltpu.async_copy(rows_v.at[0],
                                         acc.at[dst_v.at[h * FD + t]],
                                         semc, add=True)
                    for t in range(FD):
                        pltpu.make_async_copy(rows_v.at[0],
                                              acc.at[dst_v.at[h * FD + t]],
                                              semc).wait()

            plsc.subcore_barrier()
            readback(cnts_h)

        zero_acc()
        plsc.subcore_barrier()

        @pl.loop(0, CH // GRP)
        def _group(g):
            pltpu.sync_copy(src_h.at[pl.ds(idx0 + g * GRP, GRP)], src_v)
            pltpu.sync_copy(dst_h.at[pl.ds(idx0 + g * GRP, GRP)], dst_v)

            for b in range(2):
                pltpu.async_copy(feat_h.at[src_v.at[b]], rows_v.at[b], sems[b])

            @pl.loop(0, GRP // 2 - 1)
            def _pipe(q):
                j = 2 * q
                for b in range(2):
                    pltpu.make_async_copy(feat_h.at[src_v.at[j + b]],
                                          rows_v.at[b], sems[b]).wait()
                    pltpu.sync_copy(rows_v.at[b], acc.at[dst_v.at[j + b]],
                                    add=True)
                    pltpu.async_copy(feat_h.at[src_v.at[j + 2 + b]],
                                     rows_v.at[b], sems[b])

            jl = GRP - 2
            for b in range(2):
                pltpu.make_async_copy(feat_h.at[src_v.at[jl + b]],
                                      rows_v.at[b], sems[b]).wait()
                pltpu.sync_copy(rows_v.at[b], acc.at[dst_v.at[jl + b]],
                                add=True)

        plsc.subcore_barrier()
        readback(sums_h)

    return run(feat, src_all, dst_all, own_idx, zrows, ones_c)


def _tc_layer(sums, cnts, Wa, ba, Wb, bb, relu):
    NPAD, D = sums.shape[1], sums.shape[2]

    def body(s_ref, c_ref, wa_ref, ba_ref, wb_ref, bb_ref, o_ref):
        ca = c_ref[0][:, 0:1]
        cb = c_ref[1][:, 0:1]
        ma = s_ref[0] / jnp.maximum(ca, 1.0)
        mb = s_ref[1] / jnp.maximum(cb, 1.0)
        h = (jnp.dot(ma, wa_ref[...], preferred_element_type=jnp.float32)
             + jnp.dot(mb, wb_ref[...], preferred_element_type=jnp.float32)
             + jnp.where(ca > 0, 1.0, 0.0) * ba_ref[...]
             + jnp.where(cb > 0, 1.0, 0.0) * bb_ref[...])
        if relu:
            h = jnp.where(h >= 0, h, 0.01 * h)
        o_ref[...] = h

    return pl.pallas_call(
        body,
        out_shape=jax.ShapeDtypeStruct((NPAD, D), jnp.float32),
    )(sums, cnts, Wa, ba.reshape(1, D), Wb, bb.reshape(1, D))


def kernel(x, edge_index_a, edge_index_b, W1_a, b1_a, W1_b, b1_b,
           W2_a, b2_a, W2_b, b2_b):
    N, D = x.shape
    E = edge_index_a.shape[1]
    NPAD = -(-(N + 1) // (NS * 8)) * (NS * 8)
    per_tile = -(-E // (NS * CHUNK * GRP)) * CHUNK * GRP
    CH = per_tile // CHUNK
    EPAD = per_tile * NS

    def pad_idx(v):
        return jnp.concatenate(
            [v, jnp.full((EPAD - E,), N, jnp.int32)]).reshape(NS * CH, CHUNK)

    src_all = jnp.concatenate([pad_idx(edge_index_a[0]), pad_idx(edge_index_b[0])])
    dst_all = jnp.concatenate([pad_idx(edge_index_a[1]), pad_idx(edge_index_b[1])])

    x_pad = jnp.pad(x, ((0, NPAD - N), (0, 0)))
    zrows = jnp.zeros((CHUNK, D), jnp.float32)
    ones_c = jnp.ones((CHUNK, D), jnp.float32)

    sums1, cnts = _sc_agg(x_pad, src_all, dst_all, zrows, ones_c, True)
    sums1 = sums1.reshape(NC, NPAD, D)
    cnts = cnts.reshape(NC, NPAD, D)
    h = _tc_layer(sums1, cnts, W1_a, b1_a, W1_b, b1_b, relu=True)
    res2 = _sc_agg(h, src_all, dst_all, zrows, ones_c, False)
    sums2 = (res2[0] if isinstance(res2, (tuple, list)) else res2)
    sums2 = sums2.reshape(NC, NPAD, D)
    out = _tc_layer(sums2, cnts, W2_a, b2_a, W2_b, b2_b, relu=False)
    return out[:N]
